# Initial kernel scaffold; baseline (speedup 1.0000x reference)
import jax
import jax.numpy as jnp
from jax.experimental import pallas as pl


def kernel(x):
    raise NotImplementedError("write your pallas kernel here")


# speedup vs baseline: 26.4947x; 26.4947x over previous
import functools

import jax
import jax.numpy as jnp
from jax.experimental import pallas as pl
from jax.experimental.pallas import tpu as pltpu

N = 8192
D = 256
K = 5
EPS = 1e-08
BLOCK_R = 256


def _loss_kernel(x_ref, out_ref, xn_ref):
    i = pl.program_id(0)

    @pl.when(i == 0)
    def _init():
        xw = x_ref[...]
        norm = jnp.sqrt(jnp.sum(xw * xw, axis=1, keepdims=True))
        xn_ref[...] = xw / jnp.maximum(norm, 1e-12)
        out_ref[...] = jnp.zeros((1, 1), jnp.float32)

    a = xn_ref[pl.ds(i * BLOCK_R, BLOCK_R), :]
    dots = jax.lax.dot_general(
        a,
        xn_ref[...],
        dimension_numbers=(((1,), (1,)), ((), ())),
        preferred_element_type=jnp.float32,
    )

    col = jax.lax.broadcasted_iota(jnp.int32, (BLOCK_R, N), 1)
    row = jax.lax.broadcasted_iota(jnp.int32, (BLOCK_R, N), 0) + i * BLOCK_R
    dots = jnp.where(col == row, -3.0, dots)

    acc = jnp.zeros((BLOCK_R, 1), jnp.float32)
    d = dots
    for k in range(K):
        m = jnp.max(d, axis=1, keepdims=True)
        acc = acc + jnp.sqrt(jnp.maximum(2.0 - 2.0 * m, 0.0))
        if k != K - 1:
            d = jnp.where(d == m, -3.0, d)

    mean_rho = acc * (1.0 / K)
    out_ref[...] += jnp.sum(jnp.log(mean_rho + EPS)).reshape(1, 1)


@jax.jit
def kernel(x):
    total = pl.pallas_call(
        _loss_kernel,
        grid=(N // BLOCK_R,),
        in_specs=[pl.BlockSpec((N, D), lambda i: (0, 0))],
        out_specs=pl.BlockSpec((1, 1), lambda i: (0, 0)),
        out_shape=jax.ShapeDtypeStruct((1, 1), jnp.float32),
        scratch_shapes=[pltpu.VMEM((N, D), jnp.float32)],
    )(x)
    return -total[0, 0] / N


# speedup vs baseline: 31.7727x; 1.1992x over previous
import functools

import jax
import jax.numpy as jnp
from jax.experimental import pallas as pl
from jax.experimental.pallas import tpu as pltpu

N = 8192
D = 256
K = 5
EPS = 1e-08
BLOCK_R = 256


def _loss_kernel(x_ref, out_ref, xn_ref):
    i = pl.program_id(0)

    @pl.when(i == 0)
    def _init():
        xw = x_ref[...]
        norm = jnp.sqrt(jnp.sum(xw * xw, axis=1, keepdims=True))
        xn_ref[...] = xw / jnp.maximum(norm, 1e-12)
        out_ref[...] = jnp.zeros((1, 1), jnp.float32)

    a = xn_ref[pl.ds(i * BLOCK_R, BLOCK_R), :]
    dots = jax.lax.dot_general(
        a,
        xn_ref[...],
        dimension_numbers=(((1,), (1,)), ((), ())),
        preferred_element_type=jnp.float32,
    )

    col = jax.lax.broadcasted_iota(jnp.int32, (BLOCK_R, N), 1)
    row = jax.lax.broadcasted_iota(jnp.int32, (BLOCK_R, N), 0) + i * BLOCK_R
    dots = jnp.where(col == row, -3.0, dots)

    CW = 128
    neg = jnp.full((BLOCK_R, CW), -3.0, jnp.float32)
    r1 = dots[:, :CW]
    r2 = r3 = r4 = r5 = neg
    for c in range(1, N // CW):
        v = dots[:, c * CW:(c + 1) * CW]
        t1 = jnp.maximum(r1, v)
        b1 = jnp.minimum(r1, v)
        t2 = jnp.maximum(r2, b1)
        b2 = jnp.minimum(r2, b1)
        t3 = jnp.maximum(r3, b2)
        b3 = jnp.minimum(r3, b2)
        t4 = jnp.maximum(r4, b3)
        b4 = jnp.minimum(r4, b3)
        t5 = jnp.maximum(r5, b4)
        r1, r2, r3, r4, r5 = t1, t2, t3, t4, t5

    acc = jnp.zeros((BLOCK_R, 1), jnp.float32)
    for k in range(K):
        m = jnp.max(r1, axis=1, keepdims=True)
        acc = acc + jnp.sqrt(jnp.maximum(2.0 - 2.0 * m, 0.0))
        if k != K - 1:
            hit = r1 == m
            r1 = jnp.where(hit, r2, r1)
            r2 = jnp.where(hit, r3, r2)
            r3 = jnp.where(hit, r4, r3)
            r4 = jnp.where(hit, r5, r4)
            r5 = jnp.where(hit, -3.0, r5)

    mean_rho = acc * (1.0 / K)
    out_ref[...] += jnp.sum(jnp.log(mean_rho + EPS)).reshape(1, 1)


@jax.jit
def kernel(x):
    total = pl.pallas_call(
        _loss_kernel,
        grid=(N // BLOCK_R,),
        in_specs=[pl.BlockSpec((N, D), lambda i: (0, 0))],
        out_specs=pl.BlockSpec((1, 1), lambda i: (0, 0)),
        out_shape=jax.ShapeDtypeStruct((1, 1), jnp.float32),
        scratch_shapes=[pltpu.VMEM((N, D), jnp.float32)],
    )(x)
    return -total[0, 0] / N


# speedup vs baseline: 32.7996x; 1.0323x over previous
import functools

import jax
import jax.numpy as jnp
from jax.experimental import pallas as pl
from jax.experimental.pallas import tpu as pltpu

N = 8192
D = 256
K = 5
EPS = 1e-08
BLOCK_R = 512


def _loss_kernel(x_ref, out_ref, xn_ref):
    i = pl.program_id(0)

    @pl.when(i == 0)
    def _init():
        xw = x_ref[...]
        norm = jnp.sqrt(jnp.sum(xw * xw, axis=1, keepdims=True))
        xn_ref[...] = xw / jnp.maximum(norm, 1e-12)
        out_ref[...] = jnp.zeros((1, 1), jnp.float32)

    a = xn_ref[pl.ds(i * BLOCK_R, BLOCK_R), :]
    dots = jax.lax.dot_general(
        a,
        xn_ref[...],
        dimension_numbers=(((1,), (1,)), ((), ())),
        preferred_element_type=jnp.float32,
    )

    col = jax.lax.broadcasted_iota(jnp.int32, (BLOCK_R, N), 1)
    row = jax.lax.broadcasted_iota(jnp.int32, (BLOCK_R, N), 0) + i * BLOCK_R
    dots = jnp.where(col == row, -3.0, dots)

    CW = 128
    neg = jnp.full((BLOCK_R, CW), -3.0, jnp.float32)
    r1 = dots[:, :CW]
    r2 = r3 = r4 = r5 = neg
    for c in range(1, N // CW):
        v = dots[:, c * CW:(c + 1) * CW]
        t1 = jnp.maximum(r1, v)
        b1 = jnp.minimum(r1, v)
        t2 = jnp.maximum(r2, b1)
        b2 = jnp.minimum(r2, b1)
        t3 = jnp.maximum(r3, b2)
        b3 = jnp.minimum(r3, b2)
        t4 = jnp.maximum(r4, b3)
        b4 = jnp.minimum(r4, b3)
        t5 = jnp.maximum(r5, b4)
        r1, r2, r3, r4, r5 = t1, t2, t3, t4, t5

    acc = jnp.zeros((BLOCK_R, 1), jnp.float32)
    for k in range(K):
        m = jnp.max(r1, axis=1, keepdims=True)
        acc = acc + jnp.sqrt(jnp.maximum(2.0 - 2.0 * m, 0.0))
        if k != K - 1:
            hit = r1 == m
            r1 = jnp.where(hit, r2, r1)
            r2 = jnp.where(hit, r3, r2)
            r3 = jnp.where(hit, r4, r3)
            r4 = jnp.where(hit, r5, r4)
            r5 = jnp.where(hit, -3.0, r5)

    mean_rho = acc * (1.0 / K)
    out_ref[...] += jnp.sum(jnp.log(mean_rho + EPS)).reshape(1, 1)


@jax.jit
def kernel(x):
    total = pl.pallas_call(
        _loss_kernel,
        grid=(N // BLOCK_R,),
        in_specs=[pl.BlockSpec((N, D), lambda i: (0, 0))],
        out_specs=pl.BlockSpec((1, 1), lambda i: (0, 0)),
        out_shape=jax.ShapeDtypeStruct((1, 1), jnp.float32),
        scratch_shapes=[pltpu.VMEM((N, D), jnp.float32)],
    )(x)
    return -total[0, 0] / N
